# baseline (speedup 1.0000x reference)
import functools

import jax
import jax.numpy as jnp
from jax import lax
from jax.experimental import pallas as pl
from jax.experimental.pallas import tpu as pltpu
from jax.experimental.pallas import tpu_sc as plsc

_N_VOCAB = 49408
_N_EMBD = 768
_N_TOKEN = 77
_BATCH = 1024

_NC = 2
_NS = 16
_NW = _NC * _NS
_BPW = _BATCH // _NW
_KB = 8
_SPB = _BPW // _KB
_NCH = _N_TOKEN * _SPB
_NBUF = 7
_LANES = 16
_COLV = _N_EMBD // _LANES


def _embed_body(tok_hbm, table_hbm, pos_hbm, out_hbm,
                idx_v, pos_v, buf0, buf1, buf2, buf3, buf4, buf5, buf6,
                gsem0, gsem1, gsem2, gsem3, gsem4, gsem5, gsem6,
                ssem0, ssem1, ssem2, ssem3, ssem4, ssem5, ssem6):
    c = lax.axis_index("c")
    s = lax.axis_index("s")
    wid = s * _NC + c
    b_base = wid * _BPW
    col = (wid % 4) * _BPW

    pltpu.sync_copy(tok_hbm.at[:, pl.ds((wid // 4) * 128, 128)], idx_v)

    bufs = (buf0, buf1, buf2, buf3, buf4, buf5, buf6)
    gsems = (gsem0, gsem1, gsem2, gsem3, gsem4, gsem5, gsem6)
    ssems = (ssem0, ssem1, ssem2, ssem3, ssem4, ssem5, ssem6)

    def _split(i):
        t = i // _SPB
        bb = pl.multiple_of((i % _SPB) * _KB, _KB)
        return t, bb

    def _start_gather(i, b):
        t, bb = _split(i)
        pltpu.async_copy(table_hbm.at[idx_v.at[t, pl.ds(col + bb, _KB)]],
                         bufs[b], gsems[b])

    def _wait_gather(b):
        pltpu.make_async_copy(table_hbm.at[idx_v.at[0, pl.ds(0, _KB)]],
                              bufs[b], gsems[b]).wait()

    def _start_scatter(i, b):
        t, bb = _split(i)
        pltpu.async_copy(bufs[b], out_hbm.at[t, pl.ds(b_base + bb, _KB)],
                         ssems[b])

    def _wait_scatter(b):
        pltpu.make_async_copy(bufs[b], out_hbm.at[0, pl.ds(0, _KB)],
                              ssems[b]).wait()

    for b in range(_NBUF):
        _start_gather(b, b)

    pltpu.sync_copy(pos_hbm, pos_v)

    def _zc_row(r, acc):
        def _zc_col(cc, a):
            return jnp.maximum(a, jnp.abs(pos_v[r, pl.ds(cc * _LANES, _LANES)]))
        return lax.fori_loop(0, _COLV, _zc_col, acc)
    acc = lax.fori_loop(0, _N_TOKEN, _zc_row, jnp.zeros((_LANES,), jnp.float32))
    m = acc[0]
    for j in range(1, _LANES):
        m = jnp.maximum(m, acc[j])
    pos_nonzero = m != 0.0

    def _group(p, carry):
        for b in range(_NBUF):
            i = p * _NBUF + b
            _wait_gather(b)

            @pl.when(pos_nonzero)
            def _add():
                t, _ = _split(i)
                def _col(col, __):
                    sl = pl.ds(col * _LANES, _LANES)
                    pv = pos_v[t, sl]
                    def _row(j, ___):
                        bufs[b][j, sl] = bufs[b][j, sl] + pv
                        return 0
                    return lax.fori_loop(0, _KB, _row, 0)
                lax.fori_loop(0, _COLV, _col, 0)

            _start_scatter(i, b)

            @pl.when(i + _NBUF < _NCH)
            def _next():
                _wait_scatter(b)
                _start_gather(i + _NBUF, b)
        return carry

    lax.fori_loop(0, _NCH // _NBUF, _group, 0)

    for b in range(_NBUF):
        _wait_scatter(b)


_embed = functools.partial(
    pl.kernel,
    out_type=jax.ShapeDtypeStruct((_N_TOKEN, _BATCH, _N_EMBD), jnp.float32),
    mesh=plsc.VectorSubcoreMesh(core_axis_name="c", subcore_axis_name="s"),
    scratch_types=[
        pltpu.VMEM((_N_TOKEN, 128), jnp.int32),
        pltpu.VMEM((80, _N_EMBD), jnp.float32),
        pltpu.VMEM((_KB, _N_EMBD), jnp.float32),
        pltpu.VMEM((_KB, _N_EMBD), jnp.float32),
        pltpu.VMEM((_KB, _N_EMBD), jnp.float32),
        pltpu.VMEM((_KB, _N_EMBD), jnp.float32),
        pltpu.VMEM((_KB, _N_EMBD), jnp.float32),
        pltpu.VMEM((_KB, _N_EMBD), jnp.float32),
        pltpu.VMEM((_KB, _N_EMBD), jnp.float32),
    ] + [pltpu.SemaphoreType.DMA] * 14,
)(_embed_body)


def kernel(tokens, token_embedding, position_embedding):
    tok_t = tokens.astype(jnp.int32).T
    pos = jnp.pad(position_embedding, ((0, 80 - _N_TOKEN), (0, 0)))
    out_t = _embed(tok_t, token_embedding, pos)
    return out_t.transpose(1, 0, 2)


# speedup vs baseline: 1.0002x; 1.0002x over previous
import functools

import jax
import jax.numpy as jnp
from jax import lax
from jax.experimental import pallas as pl
from jax.experimental.pallas import tpu as pltpu
from jax.experimental.pallas import tpu_sc as plsc

_N_VOCAB = 49408
_N_EMBD = 768
_N_TOKEN = 77
_BATCH = 1024

_NC = 2
_NS = 16
_NW = _NC * _NS
_BPW = _BATCH // _NW
_KB = 8
_SPB = _BPW // _KB
_NCH = _N_TOKEN * _SPB
_NBUF = 7
_LANES = 16
_COLV = _N_EMBD // _LANES


def _embed_body(tok_hbm, table_hbm, pos_hbm, out_hbm,
                idx_v, pos_v, buf0, buf1, buf2, buf3, buf4, buf5, buf6,
                gsem0, gsem1, gsem2, gsem3, gsem4, gsem5, gsem6,
                ssem0, ssem1, ssem2, ssem3, ssem4, ssem5, ssem6):
    c = lax.axis_index("c")
    s = lax.axis_index("s")
    wid = s * _NC + c
    b_base = wid * _BPW
    col = (wid % 4) * _BPW

    pltpu.sync_copy(tok_hbm.at[:, pl.ds((wid // 4) * 128, 128)], idx_v)

    bufs = (buf0, buf1, buf2, buf3, buf4, buf5, buf6)
    gsems = (gsem0, gsem1, gsem2, gsem3, gsem4, gsem5, gsem6)
    ssems = (ssem0, ssem1, ssem2, ssem3, ssem4, ssem5, ssem6)

    def _split(i):
        t = i // _SPB
        bb = pl.multiple_of((i % _SPB) * _KB, _KB)
        return t, bb

    def _start_gather(i, b):
        t, bb = _split(i)
        pltpu.async_copy(table_hbm.at[idx_v.at[t, pl.ds(col + bb, _KB)]],
                         bufs[b], gsems[b])

    def _wait_gather(b):
        pltpu.make_async_copy(table_hbm.at[idx_v.at[0, pl.ds(0, _KB)]],
                              bufs[b], gsems[b]).wait()

    def _start_scatter(i, b):
        t, bb = _split(i)
        pltpu.async_copy(bufs[b], out_hbm.at[t, pl.ds(b_base + bb, _KB)],
                         ssems[b])

    def _wait_scatter(b):
        pltpu.make_async_copy(bufs[b], out_hbm.at[0, pl.ds(0, _KB)],
                              ssems[b]).wait()

    for b in range(_NBUF):
        _start_gather(b, b)

    pltpu.sync_copy(pos_hbm, pos_v)

    def _zc_row(r, acc):
        def _zc_col(cc, a):
            return jnp.maximum(a, jnp.abs(pos_v[r, pl.ds(cc * _LANES, _LANES)]))
        return lax.fori_loop(0, _COLV, _zc_col, acc)
    acc = lax.fori_loop(0, _N_TOKEN, _zc_row, jnp.zeros((_LANES,), jnp.float32))
    m = acc[0]
    for j in range(1, _LANES):
        m = jnp.maximum(m, acc[j])
    pos_nonzero = m != 0.0

    def _group(p, carry):
        for b in range(_NBUF):
            i = p * _NBUF + b
            _wait_gather(b)

            @pl.when(pos_nonzero)
            def _add():
                t, _ = _split(i)
                def _addcol(cc, __):
                    sl = pl.ds(cc * _LANES, _LANES)
                    pv = pos_v[t, sl]
                    def _addrow(j, ___):
                        bufs[b][j, sl] = bufs[b][j, sl] + pv
                        return 0
                    return lax.fori_loop(0, _KB, _addrow, 0)
                lax.fori_loop(0, _COLV, _addcol, 0)

            _start_scatter(i, b)

            @pl.when(i + _NBUF < _NCH)
            def _next():
                _wait_scatter(b)
                _start_gather(i + _NBUF, b)
        return carry

    lax.fori_loop(0, _NCH // _NBUF, _group, 0)

    for b in range(_NBUF):
        _wait_scatter(b)


_embed = functools.partial(
    pl.kernel,
    out_type=jax.ShapeDtypeStruct((_N_TOKEN, _BATCH, _N_EMBD), jnp.float32),
    mesh=plsc.VectorSubcoreMesh(core_axis_name="c", subcore_axis_name="s"),
    scratch_types=[
        pltpu.VMEM((_N_TOKEN, 128), jnp.int32),
        pltpu.VMEM((80, _N_EMBD), jnp.float32),
        pltpu.VMEM((_KB, _N_EMBD), jnp.float32),
        pltpu.VMEM((_KB, _N_EMBD), jnp.float32),
        pltpu.VMEM((_KB, _N_EMBD), jnp.float32),
        pltpu.VMEM((_KB, _N_EMBD), jnp.float32),
        pltpu.VMEM((_KB, _N_EMBD), jnp.float32),
        pltpu.VMEM((_KB, _N_EMBD), jnp.float32),
        pltpu.VMEM((_KB, _N_EMBD), jnp.float32),
    ] + [pltpu.SemaphoreType.DMA] * 14,
)(_embed_body)


def kernel(tokens, token_embedding, position_embedding):
    tok_t = tokens.astype(jnp.int32).T
    pos = jnp.pad(position_embedding, ((0, 80 - _N_TOKEN), (0, 0)))
    out_t = _embed(tok_t, token_embedding, pos)
    return out_t.transpose(1, 0, 2)
